# baseline (speedup 1.0000x reference)
import functools

import jax
import jax.numpy as jnp
from jax import lax
from jax.experimental import pallas as pl
from jax.experimental.pallas import tpu as pltpu
from jax.experimental.pallas import tpu_sc as plsc

N = 10000
E = 320000
SBIN = 64
D = 128

NC = 2
NS = 16
NW = NC * NS
EPT = E // NW
G = 50
CH = 200
NG = CH // G
NCH = EPT // CH
RPT = 624
RPT_LAST = N - 15 * RPT

_mesh = plsc.VectorSubcoreMesh(core_axis_name="c", subcore_axis_name="s")


def _prep_body(inv_ref, pos_ref, nrm_ref, w1_ref, w2_ref, a_ref, b_ref):
    inv = inv_ref[...]
    pos = pos_ref[...]
    nrm = nrm_ref[...]
    n2 = jnp.sum(nrm * nrm, axis=1, keepdims=True)
    blk = pos.shape[0]
    z = jnp.zeros((blk, D - 7), jnp.float32)
    p1 = jnp.dot(inv, w1_ref[...], preferred_element_type=jnp.float32)
    p2 = jnp.dot(inv, w2_ref[...], preferred_element_type=jnp.float32)
    a_ref[...] = jnp.concatenate([p1, -pos, -nrm, n2, z], axis=1)
    b_ref[...] = jnp.concatenate([p2, pos, nrm, n2, z], axis=1)


def _prep(inv_feat, pos, normal, W1, W2):
    blk = 2000
    return pl.pallas_call(
        _prep_body,
        grid=(N // blk,),
        in_specs=[
            pl.BlockSpec((blk, D), lambda i: (i, 0)),
            pl.BlockSpec((blk, 3), lambda i: (i, 0)),
            pl.BlockSpec((blk, 3), lambda i: (i, 0)),
            pl.BlockSpec((D, D), lambda i: (0, 0)),
            pl.BlockSpec((D, D), lambda i: (0, 0)),
        ],
        out_specs=[
            pl.BlockSpec((blk, 2 * D), lambda i: (i, 0)),
            pl.BlockSpec((blk, 2 * D), lambda i: (i, 0)),
        ],
        out_shape=[
            jax.ShapeDtypeStruct((N, 2 * D), jnp.float32),
            jax.ShapeDtypeStruct((N, 2 * D), jnp.float32),
        ],
    )(inv_feat, pos, normal, W1, W2)


CHG = 80
GG = 80
NGG = CHG // GG
NCHG = EPT // CHG


@functools.partial(
    pl.kernel,
    mesh=_mesh,
    out_type=[
        jax.ShapeDtypeStruct((E, D), jnp.float32),
        jax.ShapeDtypeStruct((E, 16), jnp.float32),
    ],
    scratch_types=[
        pltpu.VMEM((NGG, GG), jnp.int32),
        pltpu.VMEM((NGG, GG), jnp.int32),
        pltpu.VMEM((CHG, 2 * D), jnp.float32),
        pltpu.VMEM((CHG, 2 * D), jnp.float32),
        pltpu.VMEM((CHG, D), jnp.float32),
        pltpu.VMEM((CHG, 16), jnp.float32),
        pltpu.SemaphoreType.DMA,
    ],
)
def _sc_gather(a_hbm, b_hbm, src_hbm, dst_hbm,
               gp_hbm, gt_hbm, sidx, didx, bufa, bufb, bufo, gtbuf, sem):
    c = lax.axis_index("c")
    s = lax.axis_index("s")
    wid = s * NC + c
    ebase = wid * EPT

    def chunk(k, carry):
        pltpu.sync_copy(src_hbm.at[wid].at[k], sidx)
        pltpu.sync_copy(dst_hbm.at[wid].at[k], didx)
        descs = []
        for j in range(NGG):
            sl = pl.ds(j * GG, GG)
            descs.append(pltpu.async_copy(a_hbm.at[sidx.at[j]], bufa.at[sl], sem))
            descs.append(pltpu.async_copy(b_hbm.at[didx.at[j]], bufb.at[sl], sem))
        for d_ in descs:
            d_.wait()

        def addrow(r, _):
            for q in range(8):
                ql = pl.ds(q * 16, 16)
                bufo[r, ql] = bufa[r, ql] + bufb[r, ql]
            tl = pl.ds(D, 16)
            gtbuf[r, :] = bufa[r, tl] + bufb[r, tl]
            return 0

        lax.fori_loop(0, CHG, addrow, 0)
        erow = pl.ds(ebase + k * CHG, CHG)
        pltpu.sync_copy(bufo, gp_hbm.at[erow])
        pltpu.sync_copy(gtbuf, gt_hbm.at[erow])
        return carry

    lax.fori_loop(0, NCHG, chunk, 0)


def _edge_body(gp_ref, gt_ref, ef_ref, dst_ref, w3_ref, ruv_ref, rcos_ref,
               crow_ref, wpos_ref, wnorm_ref, geo_ref, small_ref):
    gp = gp_ref[...]
    gt = gt_ref[...]
    psub = gt[:, 0:3]
    nsub = gt[:, 3:6]
    sn2 = gt[:, 6:7]
    uv = jnp.sqrt(jnp.sum(psub * psub, axis=1, keepdims=True))
    cosv = 0.5 * (sn2 - jnp.sum(nsub * nsub, axis=1, keepdims=True))
    geo = gp + jnp.dot(ef_ref[...], w3_ref[...],
                       preferred_element_type=jnp.float32)
    geo = geo + uv * ruv_ref[...] + cosv * rcos_ref[...] + crow_ref[...]
    geo = jnp.maximum(geo, 0.0)
    geo_ref[...] = geo
    pw = jnp.maximum(jnp.sum(geo * wpos_ref[...], axis=1, keepdims=True), 0.0)
    nw = jnp.maximum(jnp.sum(geo * wnorm_ref[...], axis=1, keepdims=True), 0.0)
    blk = gt.shape[0]
    one = jnp.ones((blk, 1), jnp.float32)
    small = jnp.concatenate(
        [psub * pw, nsub * nw, one, jnp.zeros((blk, 1), jnp.float32)], axis=1)
    lane = jax.lax.broadcasted_iota(jnp.int32, (blk, D), 1)
    t = (dst_ref[...] % 16) * 8
    acc = jnp.zeros((blk, D), jnp.float32)
    for ccol in range(8):
        acc = acc + jnp.where(lane == t + ccol, small[:, ccol:ccol + 1], 0.0)
    small_ref[...] = acc


def _edge(gp, gt, ef, dstc, W3, r_uv, r_cos, crow, wposT, wnormT):
    blk = 1000
    return pl.pallas_call(
        _edge_body,
        grid=(E // blk,),
        in_specs=[
            pl.BlockSpec((blk, D), lambda i: (i, 0)),
            pl.BlockSpec((blk, 16), lambda i: (i, 0)),
            pl.BlockSpec((blk, 16), lambda i: (i, 0)),
            pl.BlockSpec((blk, 1), lambda i: (i, 0)),
            pl.BlockSpec((16, D), lambda i: (0, 0)),
            pl.BlockSpec((1, D), lambda i: (0, 0)),
            pl.BlockSpec((1, D), lambda i: (0, 0)),
            pl.BlockSpec((1, D), lambda i: (0, 0)),
            pl.BlockSpec((1, D), lambda i: (0, 0)),
            pl.BlockSpec((1, D), lambda i: (0, 0)),
        ],
        out_specs=[
            pl.BlockSpec((blk, D), lambda i: (i, 0)),
            pl.BlockSpec((blk, D), lambda i: (i, 0)),
        ],
        out_shape=[
            jax.ShapeDtypeStruct((E, D), jnp.float32),
            jax.ShapeDtypeStruct((E, D), jnp.float32),
        ],
    )(gp, gt, ef, dstc, W3, r_uv, r_cos, crow, wposT, wnormT)


CHS = 80
NCHS = EPT // CHS
NP = 632


@functools.partial(
    pl.kernel,
    mesh=_mesh,
    out_type=[
        jax.ShapeDtypeStruct((NC, N, D), jnp.float32),
        jax.ShapeDtypeStruct((NC, NP, D), jnp.float32),
    ],
    scratch_types=[
        pltpu.VMEM((1, CHS), jnp.int32),
        pltpu.VMEM((1, CHS), jnp.int32),
        pltpu.VMEM((CHS, D), jnp.float32),
        pltpu.VMEM((CHS, D), jnp.float32),
        pltpu.VMEM_SHARED((N, D), jnp.float32),
        pltpu.VMEM_SHARED((NP, D), jnp.float32),
    ],
)
def _sc_scatter(geo_hbm, small_hbm, dst4_hbm, dst16_hbm, z1_hbm,
                agg_hbm, agg2_hbm, didx, didx2, geob, smallb, agg_sp, acc2_sp):
    c = lax.axis_index("c")
    s = lax.axis_index("s")
    wid = s * NC + c
    ebase = wid * EPT

    @pl.when(s < 15)
    def _():
        pltpu.sync_copy(z1_hbm.at[pl.ds(0, RPT)], agg_sp.at[pl.ds(s * RPT, RPT)])

    @pl.when(s == 15)
    def _():
        pltpu.sync_copy(z1_hbm, agg_sp.at[pl.ds(15 * RPT, RPT_LAST)])

    @pl.when(s == 0)
    def _():
        pltpu.sync_copy(z1_hbm.at[pl.ds(0, 320)], acc2_sp.at[pl.ds(0, 320)])

    @pl.when(s == 1)
    def _():
        pltpu.sync_copy(z1_hbm.at[pl.ds(0, 312)], acc2_sp.at[pl.ds(320, 312)])

    plsc.subcore_barrier()

    def chunk(k, carry):
        pltpu.sync_copy(dst4_hbm.at[wid].at[k], didx)
        pltpu.sync_copy(dst16_hbm.at[wid].at[k], didx2)
        erow = pl.ds(ebase + k * CHS, CHS)
        pltpu.sync_copy(geo_hbm.at[erow], geob)
        pltpu.sync_copy(small_hbm.at[erow], smallb)
        pltpu.sync_copy(geob, agg_sp.at[didx.at[0]], add=True)
        pltpu.sync_copy(smallb, acc2_sp.at[didx2.at[0]], add=True)
        return carry

    lax.fori_loop(0, NCHS, chunk, 0)
    plsc.subcore_barrier()

    @pl.when(s < 15)
    def _():
        rows = pl.ds(s * RPT, RPT)
        pltpu.sync_copy(agg_sp.at[rows], agg_hbm.at[c].at[rows])

    @pl.when(s == 15)
    def _():
        rows = pl.ds(15 * RPT, RPT_LAST)
        pltpu.sync_copy(agg_sp.at[rows], agg_hbm.at[c].at[rows])

    @pl.when(s == 0)
    def _():
        half = pl.ds(0, 320)
        pltpu.sync_copy(acc2_sp.at[half], agg2_hbm.at[c].at[half])

    @pl.when(s == 1)
    def _():
        half = pl.ds(320, 312)
        pltpu.sync_copy(acc2_sp.at[half], agg2_hbm.at[c].at[half])

    @pl.when(s < 15)
    def _():
        rows = pl.ds(s * RPT, RPT)
        pltpu.sync_copy(agg_sp.at[rows], agg_hbm.at[c].at[rows])

    @pl.when(s == 15)
    def _():
        rows = pl.ds(15 * RPT, RPT_LAST)
        pltpu.sync_copy(agg_sp.at[rows], agg_hbm.at[c].at[rows])


def _node_body(aggp_ref, inv_ref, w1_ref, w2_ref, crow_ref, o_ref):
    agg = aggp_ref[0] + aggp_ref[1]
    acc = jnp.dot(inv_ref[...], w1_ref[...], preferred_element_type=jnp.float32)
    acc += jnp.dot(agg, w2_ref[...], preferred_element_type=jnp.float32)
    o_ref[...] = jnp.maximum(acc + crow_ref[...], 0.0)


def _node(aggp, inv_feat, Wh1, Wh2, crow):
    blk = 2000
    return pl.pallas_call(
        _node_body,
        grid=(N // blk,),
        in_specs=[
            pl.BlockSpec((NC, blk, D), lambda i: (0, i, 0)),
            pl.BlockSpec((blk, D), lambda i: (i, 0)),
            pl.BlockSpec((D, D), lambda i: (0, 0)),
            pl.BlockSpec((D, D), lambda i: (0, 0)),
            pl.BlockSpec((1, D), lambda i: (0, 0)),
        ],
        out_specs=pl.BlockSpec((blk, D), lambda i: (i, 0)),
        out_shape=jax.ShapeDtypeStruct((N, D), jnp.float32),
    )(aggp, inv_feat, Wh1, Wh2, crow)


def _gat(h, W, al, ar, Wres, b, heads, dout):
    n = h.shape[0]
    feat = (h @ W).reshape(n, heads, dout)
    el = jnp.sum(feat * al[None, :, :], axis=-1)
    er = jnp.sum(feat * ar[None, :, :], axis=-1)
    e = jax.nn.leaky_relu(el[:, None, :] + er[None, :, :], negative_slope=0.2)
    alpha = jax.nn.softmax(e, axis=0)
    rst = jnp.einsum('sdh,sho->dho', alpha, feat)
    rst = rst + (h @ Wres).reshape(n, heads, dout)
    rst = rst + b.reshape(1, heads, dout)
    return jax.nn.elu(rst)


def kernel(pos, normal, inv_feat, edge_index, edge_feat, node_bin, graph_feat,
           W_e, W_pos, W_norm, W_hn, W_graph, W_escore,
           Wg1, al1, ar1, res1, b1, Wg2, al2, ar2, res2, b2):
    src4 = edge_index[0].reshape(NW, NCHG, NGG, GG).astype(jnp.int32)
    dst4 = edge_index[1].reshape(NW, NCHG, NGG, GG).astype(jnp.int32)
    dst4s = edge_index[1].reshape(NW, NCHS, 1, CHS).astype(jnp.int32)

    W1 = W_e[:D]
    W2 = W_e[D:2 * D]
    r_uv = W_e[2 * D:2 * D + 1]
    r_cos = W_e[2 * D + 1:2 * D + 2]
    W3 = W_e[2 * D + 2:2 * D + 2 + 16]
    W4 = W_e[2 * D + 2 + 16:]
    crow = graph_feat @ W4

    atab, btab = _prep(inv_feat, pos, normal, W1, W2)
    gp, gt = _sc_gather(atab, btab, src4, dst4)
    dstc = edge_index[1].reshape(E, 1).astype(jnp.int32)
    dst16s = (edge_index[1] // 16).reshape(NW, NCHS, 1, CHS).astype(jnp.int32)
    geo, small = _edge(gp, gt, edge_feat, dstc, W3, r_uv, r_cos, crow,
                       W_pos.reshape(1, D), W_norm.reshape(1, D))
    z1 = jnp.zeros((RPT_LAST, D), jnp.float32)
    aggp, smallpk = _sc_scatter(geo, small, dst4s, dst16s, z1)
    S = (smallpk[0] + smallpk[1]).reshape(NP * 16, 8)[:N]
    cnt = S[:, 6:7]
    inv_cnt = 1.0 / jnp.maximum(cnt, 1.0)
    pos_new = pos + S[:, 0:3] * inv_cnt
    nrm_new = normal + S[:, 3:6] * inv_cnt
    nrm_new = nrm_new / jnp.linalg.norm(nrm_new, axis=1, keepdims=True)

    Wh1 = W_hn[:D]
    Wh2 = W_hn[D:2 * D]
    Wh3 = W_hn[2 * D:]
    inv_new = _node(aggp, inv_feat, Wh1, Wh2, graph_feat @ Wh3)

    score = jnp.ones((E, 1), jnp.float32)

    bs = jax.ops.segment_sum(inv_new, node_bin, num_segments=SBIN)
    bc = jax.ops.segment_sum(jnp.ones((N, 1), jnp.float32), node_bin,
                             num_segments=SBIN)
    bin_mean = bs / jnp.maximum(bc, 1.0)
    h1 = _gat(bin_mean, Wg1, al1, ar1, res1, b1, 2, 128).reshape(SBIN, -1)
    h2 = _gat(h1, Wg2, al2, ar2, res2, b2, 2, 64).reshape(SBIN, -1)
    node_graph = jnp.mean(h2, axis=0, keepdims=True)
    edge_read = jnp.sum(aggp[0] + aggp[1], axis=0, keepdims=True) / float(E)
    gfeat = jnp.maximum(
        jnp.concatenate([node_graph, edge_read, graph_feat], axis=1) @ W_graph,
        0.0)
    return (pos_new, nrm_new, inv_new, geo, score, gfeat)


# speedup vs baseline: 1.2120x; 1.2120x over previous
import functools

import jax
import jax.numpy as jnp
from jax import lax
from jax.experimental import pallas as pl
from jax.experimental.pallas import tpu as pltpu
from jax.experimental.pallas import tpu_sc as plsc

N = 10000
E = 320000
SBIN = 64
D = 128

NC = 2
NS = 16
NW = NC * NS
EPT = E // NW
G = 50
CH = 200
NG = CH // G
NCH = EPT // CH
RPT = 624
RPT_LAST = N - 15 * RPT

_mesh = plsc.VectorSubcoreMesh(core_axis_name="c", subcore_axis_name="s")


def _prep_body(inv_ref, pos_ref, nrm_ref, w1_ref, w2_ref, a_ref, b_ref):
    inv = inv_ref[...]
    pos = pos_ref[...]
    nrm = nrm_ref[...]
    n2 = jnp.sum(nrm * nrm, axis=1, keepdims=True)
    blk = pos.shape[0]
    z = jnp.zeros((blk, D - 7), jnp.float32)
    p1 = jnp.dot(inv, w1_ref[...], preferred_element_type=jnp.float32)
    p2 = jnp.dot(inv, w2_ref[...], preferred_element_type=jnp.float32)
    a_ref[...] = jnp.concatenate([p1, -pos, -nrm, n2, z], axis=1)
    b_ref[...] = jnp.concatenate([p2, pos, nrm, n2, z], axis=1)


def _prep(inv_feat, pos, normal, W1, W2):
    blk = 2000
    return pl.pallas_call(
        _prep_body,
        grid=(N // blk,),
        in_specs=[
            pl.BlockSpec((blk, D), lambda i: (i, 0)),
            pl.BlockSpec((blk, 3), lambda i: (i, 0)),
            pl.BlockSpec((blk, 3), lambda i: (i, 0)),
            pl.BlockSpec((D, D), lambda i: (0, 0)),
            pl.BlockSpec((D, D), lambda i: (0, 0)),
        ],
        out_specs=[
            pl.BlockSpec((blk, 2 * D), lambda i: (i, 0)),
            pl.BlockSpec((blk, 2 * D), lambda i: (i, 0)),
        ],
        out_shape=[
            jax.ShapeDtypeStruct((N, 2 * D), jnp.float32),
            jax.ShapeDtypeStruct((N, 2 * D), jnp.float32),
        ],
    )(inv_feat, pos, normal, W1, W2)


CHG = 80
GG = 80
NCHG = EPT // CHG
DO = 160


@functools.partial(
    pl.kernel,
    mesh=_mesh,
    out_type=[
        jax.ShapeDtypeStruct((E, DO), jnp.float32),
    ],
    scratch_types=[
        pltpu.VMEM((2, GG), jnp.int32),
        pltpu.VMEM((2, GG), jnp.int32),
        pltpu.VMEM((CHG, 2 * D), jnp.float32),
        pltpu.VMEM((CHG, 2 * D), jnp.float32),
        pltpu.VMEM((CHG, 2 * D), jnp.float32),
        pltpu.VMEM((CHG, 2 * D), jnp.float32),
        pltpu.VMEM((CHG, DO), jnp.float32),
        pltpu.SemaphoreType.DMA,
        pltpu.SemaphoreType.DMA,
        pltpu.SemaphoreType.DMA,
    ],
)
def _sc_gather(a_hbm, b_hbm, srcm_hbm, dstm_hbm, srct_hbm, dstt_hbm, gpt_hbm,
               sidx, didx, bufa0, bufb0, bufa1, bufb1, bufo, semI, semA, semB):
    c = lax.axis_index("c")
    s = lax.axis_index("s")
    wid = s * NC + c
    ebase = wid * EPT

    def addrows(bufa, bufb):
        def addrow(r, _):
            for q in range(8):
                ql = pl.ds(q * 16, 16)
                bufo[r, ql] = bufa[r, ql] + bufb[r, ql]
            bufo[r, pl.ds(D, 16)] = bufa[r, pl.ds(D, 16)] + bufb[r, pl.ds(D, 16)]
            return 0

        lax.fori_loop(0, CHG, addrow, 0)

    def pair(k2, carry):
        kA = 2 * k2
        kB = kA + 1
        i1 = pltpu.async_copy(srcm_hbm.at[wid].at[k2], sidx, semI)
        i2 = pltpu.async_copy(dstm_hbm.at[wid].at[k2], didx, semI)
        i1.wait()
        i2.wait()
        dA1 = pltpu.async_copy(a_hbm.at[sidx.at[0]], bufa0, semA)
        dA2 = pltpu.async_copy(b_hbm.at[didx.at[0]], bufb0, semA)
        dB1 = pltpu.async_copy(a_hbm.at[sidx.at[1]], bufa1, semB)
        dB2 = pltpu.async_copy(b_hbm.at[didx.at[1]], bufb1, semB)
        dA1.wait()
        dA2.wait()
        addrows(bufa0, bufb0)
        pltpu.sync_copy(bufo, gpt_hbm.at[pl.ds(ebase + kA * CHG, CHG)])
        dB1.wait()
        dB2.wait()
        addrows(bufa1, bufb1)
        pltpu.sync_copy(bufo, gpt_hbm.at[pl.ds(ebase + kB * CHG, CHG)])
        return carry

    lax.fori_loop(0, NCHG // 2, pair, 0)
    kL = NCHG - 1
    pltpu.sync_copy(srct_hbm.at[wid].at[0], sidx.at[pl.ds(0, 1)])
    pltpu.sync_copy(dstt_hbm.at[wid].at[0], didx.at[pl.ds(0, 1)])
    dL1 = pltpu.async_copy(a_hbm.at[sidx.at[0]], bufa0, semA)
    dL2 = pltpu.async_copy(b_hbm.at[didx.at[0]], bufb0, semA)
    dL1.wait()
    dL2.wait()
    addrows(bufa0, bufb0)
    pltpu.sync_copy(bufo, gpt_hbm.at[pl.ds(ebase + kL * CHG, CHG)])


def _edge_body(gpt_ref, ef_ref, dst_ref, w3_ref, ruv_ref, rcos_ref,
               crow_ref, wpos_ref, wnorm_ref, geo_ref, small_ref):
    gpt = gpt_ref[...]
    gp = gpt[:, :D]
    gt = gpt[:, D:D + 16]
    psub = gt[:, 0:3]
    nsub = gt[:, 3:6]
    sn2 = gt[:, 6:7]
    uv = jnp.sqrt(jnp.sum(psub * psub, axis=1, keepdims=True))
    cosv = 0.5 * (sn2 - jnp.sum(nsub * nsub, axis=1, keepdims=True))
    geo = gp + jnp.dot(ef_ref[...], w3_ref[...],
                       preferred_element_type=jnp.float32)
    geo = geo + uv * ruv_ref[...] + cosv * rcos_ref[...] + crow_ref[...]
    geo = jnp.maximum(geo, 0.0)
    geo_ref[...] = geo
    pw = jnp.maximum(jnp.sum(geo * wpos_ref[...], axis=1, keepdims=True), 0.0)
    nw = jnp.maximum(jnp.sum(geo * wnorm_ref[...], axis=1, keepdims=True), 0.0)
    blk = gt.shape[0]
    one = jnp.ones((blk, 1), jnp.float32)
    small = jnp.concatenate(
        [psub * pw, nsub * nw, one, jnp.zeros((blk, 1), jnp.float32)], axis=1)
    lane = jax.lax.broadcasted_iota(jnp.int32, (blk, D), 1)
    t = (dst_ref[...] % 16) * 8
    acc = jnp.zeros((blk, D), jnp.float32)
    for ccol in range(8):
        acc = acc + jnp.where(lane == t + ccol, small[:, ccol:ccol + 1], 0.0)
    small_ref[...] = acc


def _edge(gpt, ef, dstc, W3, r_uv, r_cos, crow, wposT, wnormT):
    blk = 1000
    return pl.pallas_call(
        _edge_body,
        grid=(E // blk,),
        in_specs=[
            pl.BlockSpec((blk, DO), lambda i: (i, 0)),
            pl.BlockSpec((blk, 16), lambda i: (i, 0)),
            pl.BlockSpec((blk, 1), lambda i: (i, 0)),
            pl.BlockSpec((16, D), lambda i: (0, 0)),
            pl.BlockSpec((1, D), lambda i: (0, 0)),
            pl.BlockSpec((1, D), lambda i: (0, 0)),
            pl.BlockSpec((1, D), lambda i: (0, 0)),
            pl.BlockSpec((1, D), lambda i: (0, 0)),
            pl.BlockSpec((1, D), lambda i: (0, 0)),
        ],
        out_specs=[
            pl.BlockSpec((blk, D), lambda i: (i, 0)),
            pl.BlockSpec((blk, D), lambda i: (i, 0)),
        ],
        out_shape=[
            jax.ShapeDtypeStruct((E, D), jnp.float32),
            jax.ShapeDtypeStruct((E, D), jnp.float32),
        ],
    )(gpt, ef, dstc, W3, r_uv, r_cos, crow, wposT, wnormT)


CHS = 80
NCHS = EPT // CHS
NP = 632


@functools.partial(
    pl.kernel,
    mesh=_mesh,
    out_type=[
        jax.ShapeDtypeStruct((NC, N, D), jnp.float32),
        jax.ShapeDtypeStruct((NC, NP, D), jnp.float32),
    ],
    scratch_types=[
        pltpu.VMEM((1, CHS), jnp.int32),
        pltpu.VMEM((1, CHS), jnp.int32),
        pltpu.VMEM((CHS, D), jnp.float32),
        pltpu.VMEM((CHS, D), jnp.float32),
        pltpu.VMEM_SHARED((N, D), jnp.float32),
        pltpu.VMEM_SHARED((NP, D), jnp.float32),
    ],
)
def _sc_scatter(geo_hbm, small_hbm, dst4_hbm, dst16_hbm, z1_hbm,
                agg_hbm, agg2_hbm, didx, didx2, geob, smallb, agg_sp, acc2_sp):
    c = lax.axis_index("c")
    s = lax.axis_index("s")
    wid = s * NC + c
    ebase = wid * EPT

    @pl.when(s < 15)
    def _():
        pltpu.sync_copy(z1_hbm.at[pl.ds(0, RPT)], agg_sp.at[pl.ds(s * RPT, RPT)])

    @pl.when(s == 15)
    def _():
        pltpu.sync_copy(z1_hbm, agg_sp.at[pl.ds(15 * RPT, RPT_LAST)])

    @pl.when(s == 0)
    def _():
        pltpu.sync_copy(z1_hbm.at[pl.ds(0, 320)], acc2_sp.at[pl.ds(0, 320)])

    @pl.when(s == 1)
    def _():
        pltpu.sync_copy(z1_hbm.at[pl.ds(0, 312)], acc2_sp.at[pl.ds(320, 312)])

    plsc.subcore_barrier()

    def chunk(k, carry):
        pltpu.sync_copy(dst4_hbm.at[wid].at[k], didx)
        pltpu.sync_copy(dst16_hbm.at[wid].at[k], didx2)
        erow = pl.ds(ebase + k * CHS, CHS)
        pltpu.sync_copy(geo_hbm.at[erow], geob)
        pltpu.sync_copy(small_hbm.at[erow], smallb)
        pltpu.sync_copy(geob, agg_sp.at[didx.at[0]], add=True)
        pltpu.sync_copy(smallb, acc2_sp.at[didx2.at[0]], add=True)
        return carry

    lax.fori_loop(0, NCHS, chunk, 0)
    plsc.subcore_barrier()

    @pl.when(s < 15)
    def _():
        rows = pl.ds(s * RPT, RPT)
        pltpu.sync_copy(agg_sp.at[rows], agg_hbm.at[c].at[rows])

    @pl.when(s == 15)
    def _():
        rows = pl.ds(15 * RPT, RPT_LAST)
        pltpu.sync_copy(agg_sp.at[rows], agg_hbm.at[c].at[rows])

    @pl.when(s == 0)
    def _():
        half = pl.ds(0, 320)
        pltpu.sync_copy(acc2_sp.at[half], agg2_hbm.at[c].at[half])

    @pl.when(s == 1)
    def _():
        half = pl.ds(320, 312)
        pltpu.sync_copy(acc2_sp.at[half], agg2_hbm.at[c].at[half])

    @pl.when(s < 15)
    def _():
        rows = pl.ds(s * RPT, RPT)
        pltpu.sync_copy(agg_sp.at[rows], agg_hbm.at[c].at[rows])

    @pl.when(s == 15)
    def _():
        rows = pl.ds(15 * RPT, RPT_LAST)
        pltpu.sync_copy(agg_sp.at[rows], agg_hbm.at[c].at[rows])


def _node_body(aggp_ref, inv_ref, w1_ref, w2_ref, crow_ref, o_ref):
    agg = aggp_ref[0] + aggp_ref[1]
    acc = jnp.dot(inv_ref[...], w1_ref[...], preferred_element_type=jnp.float32)
    acc += jnp.dot(agg, w2_ref[...], preferred_element_type=jnp.float32)
    o_ref[...] = jnp.maximum(acc + crow_ref[...], 0.0)


def _node(aggp, inv_feat, Wh1, Wh2, crow):
    blk = 2000
    return pl.pallas_call(
        _node_body,
        grid=(N // blk,),
        in_specs=[
            pl.BlockSpec((NC, blk, D), lambda i: (0, i, 0)),
            pl.BlockSpec((blk, D), lambda i: (i, 0)),
            pl.BlockSpec((D, D), lambda i: (0, 0)),
            pl.BlockSpec((D, D), lambda i: (0, 0)),
            pl.BlockSpec((1, D), lambda i: (0, 0)),
        ],
        out_specs=pl.BlockSpec((blk, D), lambda i: (i, 0)),
        out_shape=jax.ShapeDtypeStruct((N, D), jnp.float32),
    )(aggp, inv_feat, Wh1, Wh2, crow)


def _gat(h, W, al, ar, Wres, b, heads, dout):
    n = h.shape[0]
    feat = (h @ W).reshape(n, heads, dout)
    el = jnp.sum(feat * al[None, :, :], axis=-1)
    er = jnp.sum(feat * ar[None, :, :], axis=-1)
    e = jax.nn.leaky_relu(el[:, None, :] + er[None, :, :], negative_slope=0.2)
    alpha = jax.nn.softmax(e, axis=0)
    rst = jnp.einsum('sdh,sho->dho', alpha, feat)
    rst = rst + (h @ Wres).reshape(n, heads, dout)
    rst = rst + b.reshape(1, heads, dout)
    return jax.nn.elu(rst)


def kernel(pos, normal, inv_feat, edge_index, edge_feat, node_bin, graph_feat,
           W_e, W_pos, W_norm, W_hn, W_graph, W_escore,
           Wg1, al1, ar1, res1, b1, Wg2, al2, ar2, res2, b2):
    srcf = edge_index[0].reshape(NW, EPT).astype(jnp.int32)
    dstf = edge_index[1].reshape(NW, EPT).astype(jnp.int32)
    srcm = srcf[:, :124 * GG].reshape(NW, 62, 2, GG)
    dstm = dstf[:, :124 * GG].reshape(NW, 62, 2, GG)
    srct = srcf[:, 124 * GG:].reshape(NW, 1, 1, GG)
    dstt = dstf[:, 124 * GG:].reshape(NW, 1, 1, GG)
    dst4s = edge_index[1].reshape(NW, NCHS, 1, CHS).astype(jnp.int32)

    W1 = W_e[:D]
    W2 = W_e[D:2 * D]
    r_uv = W_e[2 * D:2 * D + 1]
    r_cos = W_e[2 * D + 1:2 * D + 2]
    W3 = W_e[2 * D + 2:2 * D + 2 + 16]
    W4 = W_e[2 * D + 2 + 16:]
    crow = graph_feat @ W4

    atab, btab = _prep(inv_feat, pos, normal, W1, W2)
    (gpt,) = _sc_gather(atab, btab, srcm, dstm, srct, dstt)
    dstc = edge_index[1].reshape(E, 1).astype(jnp.int32)
    dst16s = (edge_index[1] // 16).reshape(NW, NCHS, 1, CHS).astype(jnp.int32)
    geo, small = _edge(gpt, edge_feat, dstc, W3, r_uv, r_cos, crow,
                       W_pos.reshape(1, D), W_norm.reshape(1, D))
    z1 = jnp.zeros((RPT_LAST, D), jnp.float32)
    aggp, smallpk = _sc_scatter(geo, small, dst4s, dst16s, z1)
    S = (smallpk[0] + smallpk[1]).reshape(NP * 16, 8)[:N]
    cnt = S[:, 6:7]
    inv_cnt = 1.0 / jnp.maximum(cnt, 1.0)
    pos_new = pos + S[:, 0:3] * inv_cnt
    nrm_new = normal + S[:, 3:6] * inv_cnt
    nrm_new = nrm_new / jnp.linalg.norm(nrm_new, axis=1, keepdims=True)

    Wh1 = W_hn[:D]
    Wh2 = W_hn[D:2 * D]
    Wh3 = W_hn[2 * D:]
    inv_new = _node(aggp, inv_feat, Wh1, Wh2, graph_feat @ Wh3)

    score = jnp.ones((E, 1), jnp.float32)

    bs = jax.ops.segment_sum(inv_new, node_bin, num_segments=SBIN)
    bc = jax.ops.segment_sum(jnp.ones((N, 1), jnp.float32), node_bin,
                             num_segments=SBIN)
    bin_mean = bs / jnp.maximum(bc, 1.0)
    h1 = _gat(bin_mean, Wg1, al1, ar1, res1, b1, 2, 128).reshape(SBIN, -1)
    h2 = _gat(h1, Wg2, al2, ar2, res2, b2, 2, 64).reshape(SBIN, -1)
    node_graph = jnp.mean(h2, axis=0, keepdims=True)
    edge_read = jnp.sum(aggp[0] + aggp[1], axis=0, keepdims=True) / float(E)
    gfeat = jnp.maximum(
        jnp.concatenate([node_graph, edge_read, graph_feat], axis=1) @ W_graph,
        0.0)
    return (pos_new, nrm_new, inv_new, geo, score, gfeat)


# speedup vs baseline: 1.3258x; 1.0939x over previous
import functools

import jax
import jax.numpy as jnp
from jax import lax
from jax.experimental import pallas as pl
from jax.experimental.pallas import tpu as pltpu
from jax.experimental.pallas import tpu_sc as plsc

N = 10000
E = 320000
SBIN = 64
D = 128

NC = 2
NS = 16
NW = NC * NS
EPT = E // NW
G = 50
CH = 200
NG = CH // G
NCH = EPT // CH
RPT = 624
RPT_LAST = N - 15 * RPT

_mesh = plsc.VectorSubcoreMesh(core_axis_name="c", subcore_axis_name="s")


def _prep_body(inv_ref, pos_ref, nrm_ref, w1_ref, w2_ref, a_ref, b_ref):
    inv = inv_ref[...]
    pos = pos_ref[...]
    nrm = nrm_ref[...]
    n2 = jnp.sum(nrm * nrm, axis=1, keepdims=True)
    blk = pos.shape[0]
    z = jnp.zeros((blk, D - 7), jnp.float32)
    p1 = jnp.dot(inv, w1_ref[...], preferred_element_type=jnp.float32)
    p2 = jnp.dot(inv, w2_ref[...], preferred_element_type=jnp.float32)
    a_ref[...] = jnp.concatenate([p1, -pos, -nrm, n2, z], axis=1)
    b_ref[...] = jnp.concatenate([p2, pos, nrm, n2, z], axis=1)


def _prep(inv_feat, pos, normal, W1, W2):
    blk = 2000
    return pl.pallas_call(
        _prep_body,
        grid=(N // blk,),
        in_specs=[
            pl.BlockSpec((blk, D), lambda i: (i, 0)),
            pl.BlockSpec((blk, 3), lambda i: (i, 0)),
            pl.BlockSpec((blk, 3), lambda i: (i, 0)),
            pl.BlockSpec((D, D), lambda i: (0, 0)),
            pl.BlockSpec((D, D), lambda i: (0, 0)),
        ],
        out_specs=[
            pl.BlockSpec((blk, 2 * D), lambda i: (i, 0)),
            pl.BlockSpec((blk, 2 * D), lambda i: (i, 0)),
        ],
        out_shape=[
            jax.ShapeDtypeStruct((N, 2 * D), jnp.float32),
            jax.ShapeDtypeStruct((N, 2 * D), jnp.float32),
        ],
    )(inv_feat, pos, normal, W1, W2)


CHG = 80
GG = 80
NCHG = EPT // CHG
DO = 160


@functools.partial(
    pl.kernel,
    mesh=_mesh,
    out_type=[
        jax.ShapeDtypeStruct((E, DO), jnp.float32),
    ],
    scratch_types=[
        pltpu.VMEM((2, GG), jnp.int32),
        pltpu.VMEM((2, GG), jnp.int32),
        pltpu.VMEM((CHG, 2 * D), jnp.float32),
        pltpu.VMEM((CHG, 2 * D), jnp.float32),
        pltpu.VMEM((CHG, 2 * D), jnp.float32),
        pltpu.VMEM((CHG, 2 * D), jnp.float32),
        pltpu.VMEM((CHG, DO), jnp.float32),
        pltpu.SemaphoreType.DMA,
        pltpu.SemaphoreType.DMA,
        pltpu.SemaphoreType.DMA,
    ],
)
def _sc_gather(a_hbm, b_hbm, srcm_hbm, dstm_hbm, srct_hbm, dstt_hbm, gpt_hbm,
               sidx, didx, bufa0, bufb0, bufa1, bufb1, bufo, semI, semA, semB):
    c = lax.axis_index("c")
    s = lax.axis_index("s")
    wid = s * NC + c
    ebase = wid * EPT

    def addrows(bufa, bufb):
        def addrow(r, _):
            for q in range(8):
                ql = pl.ds(q * 16, 16)
                bufo[r, ql] = bufa[r, ql] + bufb[r, ql]
            bufo[r, pl.ds(D, 16)] = bufa[r, pl.ds(D, 16)] + bufb[r, pl.ds(D, 16)]
            return 0

        lax.fori_loop(0, CHG, addrow, 0)

    def pair(k2, carry):
        kA = 2 * k2
        kB = kA + 1
        i1 = pltpu.async_copy(srcm_hbm.at[wid].at[k2], sidx, semI)
        i2 = pltpu.async_copy(dstm_hbm.at[wid].at[k2], didx, semI)
        i1.wait()
        i2.wait()
        dA1 = pltpu.async_copy(a_hbm.at[sidx.at[0]], bufa0, semA)
        dA2 = pltpu.async_copy(b_hbm.at[didx.at[0]], bufb0, semA)
        dB1 = pltpu.async_copy(a_hbm.at[sidx.at[1]], bufa1, semB)
        dB2 = pltpu.async_copy(b_hbm.at[didx.at[1]], bufb1, semB)
        dA1.wait()
        dA2.wait()
        addrows(bufa0, bufb0)
        pltpu.sync_copy(bufo, gpt_hbm.at[pl.ds(ebase + kA * CHG, CHG)])
        dB1.wait()
        dB2.wait()
        addrows(bufa1, bufb1)
        pltpu.sync_copy(bufo, gpt_hbm.at[pl.ds(ebase + kB * CHG, CHG)])
        return carry

    lax.fori_loop(0, NCHG // 2, pair, 0)
    kL = NCHG - 1
    pltpu.sync_copy(srct_hbm.at[wid].at[0], sidx.at[pl.ds(0, 1)])
    pltpu.sync_copy(dstt_hbm.at[wid].at[0], didx.at[pl.ds(0, 1)])
    dL1 = pltpu.async_copy(a_hbm.at[sidx.at[0]], bufa0, semA)
    dL2 = pltpu.async_copy(b_hbm.at[didx.at[0]], bufb0, semA)
    dL1.wait()
    dL2.wait()
    addrows(bufa0, bufb0)
    pltpu.sync_copy(bufo, gpt_hbm.at[pl.ds(ebase + kL * CHG, CHG)])


def _edge_body(gpt_ref, ef_ref, dst_ref, w3_ref, ruv_ref, rcos_ref,
               crow_ref, wpos_ref, wnorm_ref, geo_ref, small_ref):
    gpt = gpt_ref[...]
    gp = gpt[:, :D]
    gt = gpt[:, D:D + 16]
    psub = gt[:, 0:3]
    nsub = gt[:, 3:6]
    sn2 = gt[:, 6:7]
    uv = jnp.sqrt(jnp.sum(psub * psub, axis=1, keepdims=True))
    cosv = 0.5 * (sn2 - jnp.sum(nsub * nsub, axis=1, keepdims=True))
    geo = gp + jnp.dot(ef_ref[...], w3_ref[...],
                       preferred_element_type=jnp.float32)
    geo = geo + uv * ruv_ref[...] + cosv * rcos_ref[...] + crow_ref[...]
    geo = jnp.maximum(geo, 0.0)
    geo_ref[...] = geo
    pw = jnp.maximum(jnp.sum(geo * wpos_ref[...], axis=1, keepdims=True), 0.0)
    nw = jnp.maximum(jnp.sum(geo * wnorm_ref[...], axis=1, keepdims=True), 0.0)
    blk = gt.shape[0]
    one = jnp.ones((blk, 1), jnp.float32)
    small = jnp.concatenate(
        [psub * pw, nsub * nw, one, jnp.zeros((blk, 1), jnp.float32)], axis=1)
    lane = jax.lax.broadcasted_iota(jnp.int32, (blk, D), 1)
    t = (dst_ref[...] % 16) * 8
    acc = jnp.zeros((blk, D), jnp.float32)
    for ccol in range(8):
        acc = acc + jnp.where(lane == t + ccol, small[:, ccol:ccol + 1], 0.0)
    small_ref[...] = acc


def _edge(gpt, ef, dstc, W3, r_uv, r_cos, crow, wposT, wnormT):
    blk = 1000
    return pl.pallas_call(
        _edge_body,
        grid=(E // blk,),
        in_specs=[
            pl.BlockSpec((blk, DO), lambda i: (i, 0)),
            pl.BlockSpec((blk, 16), lambda i: (i, 0)),
            pl.BlockSpec((blk, 1), lambda i: (i, 0)),
            pl.BlockSpec((16, D), lambda i: (0, 0)),
            pl.BlockSpec((1, D), lambda i: (0, 0)),
            pl.BlockSpec((1, D), lambda i: (0, 0)),
            pl.BlockSpec((1, D), lambda i: (0, 0)),
            pl.BlockSpec((1, D), lambda i: (0, 0)),
            pl.BlockSpec((1, D), lambda i: (0, 0)),
        ],
        out_specs=[
            pl.BlockSpec((blk, D), lambda i: (i, 0)),
            pl.BlockSpec((blk, D), lambda i: (i, 0)),
        ],
        out_shape=[
            jax.ShapeDtypeStruct((E, D), jnp.float32),
            jax.ShapeDtypeStruct((E, D), jnp.float32),
        ],
    )(gpt, ef, dstc, W3, r_uv, r_cos, crow, wposT, wnormT)


CHS = 80
NCHS = EPT // CHS
NP = 632


@functools.partial(
    pl.kernel,
    mesh=_mesh,
    out_type=[
        jax.ShapeDtypeStruct((NC, N, D), jnp.float32),
        jax.ShapeDtypeStruct((NC, NP, D), jnp.float32),
    ],
    scratch_types=[
        pltpu.VMEM((1, CHS), jnp.int32),
        pltpu.VMEM((1, CHS), jnp.int32),
        pltpu.VMEM((CHS, D), jnp.float32),
        pltpu.VMEM((CHS, D), jnp.float32),
        pltpu.VMEM_SHARED((N, D), jnp.float32),
        pltpu.VMEM_SHARED((NP, D), jnp.float32),
        pltpu.SemaphoreType.DMA,
        pltpu.SemaphoreType.DMA,
        pltpu.SemaphoreType.DMA,
    ],
)
def _sc_scatter(geo_hbm, small_hbm, dst4_hbm, dst16_hbm, z1_hbm,
                agg_hbm, agg2_hbm, didx, didx2, geob, smallb, agg_sp, acc2_sp,
                semI, semA, semC):
    c = lax.axis_index("c")
    s = lax.axis_index("s")
    wid = s * NC + c
    ebase = wid * EPT

    @pl.when(s < 15)
    def _():
        pltpu.sync_copy(z1_hbm.at[pl.ds(0, RPT)], agg_sp.at[pl.ds(s * RPT, RPT)])

    @pl.when(s == 15)
    def _():
        pltpu.sync_copy(z1_hbm, agg_sp.at[pl.ds(15 * RPT, RPT_LAST)])

    @pl.when(s == 0)
    def _():
        pltpu.sync_copy(z1_hbm.at[pl.ds(0, 320)], acc2_sp.at[pl.ds(0, 320)])

    @pl.when(s == 1)
    def _():
        pltpu.sync_copy(z1_hbm.at[pl.ds(0, 312)], acc2_sp.at[pl.ds(320, 312)])

    plsc.subcore_barrier()

    def chunk(k, carry):
        erow = pl.ds(ebase + k * CHS, CHS)
        i1 = pltpu.async_copy(dst4_hbm.at[wid].at[k], didx, semI)
        i2 = pltpu.async_copy(dst16_hbm.at[wid].at[k], didx2, semI)
        g1 = pltpu.async_copy(geo_hbm.at[erow], geob, semA)
        g2 = pltpu.async_copy(small_hbm.at[erow], smallb, semA)
        i1.wait()
        i2.wait()
        g1.wait()
        g2.wait()
        a1 = pltpu.async_copy(geob, agg_sp.at[didx.at[0]], semC, add=True)
        a2 = pltpu.async_copy(smallb, acc2_sp.at[didx2.at[0]], semC, add=True)
        a1.wait()
        a2.wait()
        return carry

    lax.fori_loop(0, NCHS, chunk, 0)
    plsc.subcore_barrier()

    @pl.when(s < 15)
    def _():
        rows = pl.ds(s * RPT, RPT)
        pltpu.sync_copy(agg_sp.at[rows], agg_hbm.at[c].at[rows])

    @pl.when(s == 15)
    def _():
        rows = pl.ds(15 * RPT, RPT_LAST)
        pltpu.sync_copy(agg_sp.at[rows], agg_hbm.at[c].at[rows])

    @pl.when(s == 0)
    def _():
        half = pl.ds(0, 320)
        pltpu.sync_copy(acc2_sp.at[half], agg2_hbm.at[c].at[half])

    @pl.when(s == 1)
    def _():
        half = pl.ds(320, 312)
        pltpu.sync_copy(acc2_sp.at[half], agg2_hbm.at[c].at[half])

    @pl.when(s < 15)
    def _():
        rows = pl.ds(s * RPT, RPT)
        pltpu.sync_copy(agg_sp.at[rows], agg_hbm.at[c].at[rows])

    @pl.when(s == 15)
    def _():
        rows = pl.ds(15 * RPT, RPT_LAST)
        pltpu.sync_copy(agg_sp.at[rows], agg_hbm.at[c].at[rows])


def _node_body(aggp_ref, inv_ref, w1_ref, w2_ref, crow_ref, o_ref):
    agg = aggp_ref[0] + aggp_ref[1]
    acc = jnp.dot(inv_ref[...], w1_ref[...], preferred_element_type=jnp.float32)
    acc += jnp.dot(agg, w2_ref[...], preferred_element_type=jnp.float32)
    o_ref[...] = jnp.maximum(acc + crow_ref[...], 0.0)


def _node(aggp, inv_feat, Wh1, Wh2, crow):
    blk = 2000
    return pl.pallas_call(
        _node_body,
        grid=(N // blk,),
        in_specs=[
            pl.BlockSpec((NC, blk, D), lambda i: (0, i, 0)),
            pl.BlockSpec((blk, D), lambda i: (i, 0)),
            pl.BlockSpec((D, D), lambda i: (0, 0)),
            pl.BlockSpec((D, D), lambda i: (0, 0)),
            pl.BlockSpec((1, D), lambda i: (0, 0)),
        ],
        out_specs=pl.BlockSpec((blk, D), lambda i: (i, 0)),
        out_shape=jax.ShapeDtypeStruct((N, D), jnp.float32),
    )(aggp, inv_feat, Wh1, Wh2, crow)


def _gat(h, W, al, ar, Wres, b, heads, dout):
    n = h.shape[0]
    feat = (h @ W).reshape(n, heads, dout)
    el = jnp.sum(feat * al[None, :, :], axis=-1)
    er = jnp.sum(feat * ar[None, :, :], axis=-1)
    e = jax.nn.leaky_relu(el[:, None, :] + er[None, :, :], negative_slope=0.2)
    alpha = jax.nn.softmax(e, axis=0)
    rst = jnp.einsum('sdh,sho->dho', alpha, feat)
    rst = rst + (h @ Wres).reshape(n, heads, dout)
    rst = rst + b.reshape(1, heads, dout)
    return jax.nn.elu(rst)


def kernel(pos, normal, inv_feat, edge_index, edge_feat, node_bin, graph_feat,
           W_e, W_pos, W_norm, W_hn, W_graph, W_escore,
           Wg1, al1, ar1, res1, b1, Wg2, al2, ar2, res2, b2):
    srcf = edge_index[0].reshape(NW, EPT).astype(jnp.int32)
    dstf = edge_index[1].reshape(NW, EPT).astype(jnp.int32)
    srcm = srcf[:, :124 * GG].reshape(NW, 62, 2, GG)
    dstm = dstf[:, :124 * GG].reshape(NW, 62, 2, GG)
    srct = srcf[:, 124 * GG:].reshape(NW, 1, 1, GG)
    dstt = dstf[:, 124 * GG:].reshape(NW, 1, 1, GG)
    dst4s = edge_index[1].reshape(NW, NCHS, 1, CHS).astype(jnp.int32)

    W1 = W_e[:D]
    W2 = W_e[D:2 * D]
    r_uv = W_e[2 * D:2 * D + 1]
    r_cos = W_e[2 * D + 1:2 * D + 2]
    W3 = W_e[2 * D + 2:2 * D + 2 + 16]
    W4 = W_e[2 * D + 2 + 16:]
    crow = graph_feat @ W4

    atab, btab = _prep(inv_feat, pos, normal, W1, W2)
    (gpt,) = _sc_gather(atab, btab, srcm, dstm, srct, dstt)
    dstc = edge_index[1].reshape(E, 1).astype(jnp.int32)
    dst16s = (edge_index[1] // 16).reshape(NW, NCHS, 1, CHS).astype(jnp.int32)
    geo, small = _edge(gpt, edge_feat, dstc, W3, r_uv, r_cos, crow,
                       W_pos.reshape(1, D), W_norm.reshape(1, D))
    z1 = jnp.zeros((RPT_LAST, D), jnp.float32)
    aggp, smallpk = _sc_scatter(geo, small, dst4s, dst16s, z1)
    S = (smallpk[0] + smallpk[1]).reshape(NP * 16, 8)[:N]
    cnt = S[:, 6:7]
    inv_cnt = 1.0 / jnp.maximum(cnt, 1.0)
    pos_new = pos + S[:, 0:3] * inv_cnt
    nrm_new = normal + S[:, 3:6] * inv_cnt
    nrm_new = nrm_new / jnp.linalg.norm(nrm_new, axis=1, keepdims=True)

    Wh1 = W_hn[:D]
    Wh2 = W_hn[D:2 * D]
    Wh3 = W_hn[2 * D:]
    inv_new = _node(aggp, inv_feat, Wh1, Wh2, graph_feat @ Wh3)

    score = jnp.ones((E, 1), jnp.float32)

    bs = jax.ops.segment_sum(inv_new, node_bin, num_segments=SBIN)
    bc = jax.ops.segment_sum(jnp.ones((N, 1), jnp.float32), node_bin,
                             num_segments=SBIN)
    bin_mean = bs / jnp.maximum(bc, 1.0)
    h1 = _gat(bin_mean, Wg1, al1, ar1, res1, b1, 2, 128).reshape(SBIN, -1)
    h2 = _gat(h1, Wg2, al2, ar2, res2, b2, 2, 64).reshape(SBIN, -1)
    node_graph = jnp.mean(h2, axis=0, keepdims=True)
    edge_read = jnp.sum(aggp[0] + aggp[1], axis=0, keepdims=True) / float(E)
    gfeat = jnp.maximum(
        jnp.concatenate([node_graph, edge_read, graph_feat], axis=1) @ W_graph,
        0.0)
    return (pos_new, nrm_new, inv_new, geo, score, gfeat)
